# baseline (speedup 1.0000x reference)
import functools
import math

import jax
import jax.numpy as jnp
from jax import lax
from jax.experimental import pallas as pl
from jax.experimental.pallas import tpu as pltpu
from jax.experimental.pallas import tpu_sc as plsc

_LS = 0.1
_V = 10000
_PAD = 0
_CONF = 1.0 - _LS
_SMOOTH = _LS / (_V - 2)
_C_ROW = (_V - 2) * _SMOOTH * math.log(_SMOOTH) + _CONF * math.log(_CONF)

_VB = 216
_VS = 8208
_NC, _NS, _L = 2, 16, 16
_NW = _NC * _NS


def _tc_body(t_ref, x_ref, o_ref):
    b = pl.program_id(0)
    x = x_ref[...]
    t = t_ref[...]
    colsum = jnp.sum(x, axis=0, keepdims=True)
    rows = jax.lax.broadcasted_iota(jnp.int32, x.shape, 0) + b * _VB
    tval = jnp.sum(jnp.where(rows == t, x, 0.0), axis=0, keepdims=True)
    contrib = -_SMOOTH * colsum - (_CONF - _SMOOTH) * tval
    head = jnp.where(b == 0, _C_ROW + _SMOOTH * x[0:1, :], 0.0)
    part = jnp.sum(jnp.where(t != _PAD, contrib + head, 0.0))

    @pl.when(b == 0)
    def _():
        o_ref[0, 0] = 0.0

    o_ref[0, 0] += part


_SLAB_R = 8
_SLAB_C = 4096


def _sc_body(n, v, xt_hbm, tgt_hbm, out_hbm, tvm, buf0, buf1, accv, sem0, sem1):
    rw = (v - _VS) // _NW
    ng = rw // _SLAB_R
    nh = n // _SLAB_C
    wid = lax.axis_index("s") * _NC + lax.axis_index("c")
    base = _VS + wid * rw
    pltpu.sync_copy(tgt_hbm, tvm)

    def hs_src(g, h):
        return xt_hbm.at[pl.ds(base + g * _SLAB_R, _SLAB_R),
                         pl.ds(h * _SLAB_C, _SLAB_C)]

    def compute(buf, g, co, acc):
        rbase = base + g * _SLAB_R

        def vec_step(k, kacc):
            sl = pl.ds(pl.multiple_of(k * _L, _L), _L)
            tk = tvm[pl.ds(pl.multiple_of(co + k * _L, _L), _L)]
            f0 = jnp.where(tk != _PAD, -_SMOOTH, 0.0)
            for rr in range(_SLAB_R):
                xv = buf[rr, sl]
                f = jnp.where(tk == rbase + rr, -_CONF, f0)
                kacc = kacc + xv * f
            return kacc

        return lax.fori_loop(0, _SLAB_C // _L, vec_step, acc)

    pltpu.async_copy(hs_src(0, 0), buf0, sem0)

    def pair_step(i, acc):
        g0 = (2 * i) // nh
        h0 = (2 * i) % nh
        g1 = (2 * i + 1) // nh
        h1 = (2 * i + 1) % nh
        g2 = jnp.minimum((2 * i + 2) // nh, ng - 1)
        h2 = (2 * i + 2) % nh
        pltpu.async_copy(hs_src(g1, h1), buf1, sem1)
        pltpu.make_async_copy(hs_src(g0, h0), buf0, sem0).wait()
        acc = compute(buf0, g0, h0 * _SLAB_C, acc)
        pltpu.async_copy(hs_src(g2, h2), buf0, sem0)
        pltpu.make_async_copy(hs_src(g1, h1), buf1, sem1).wait()
        return compute(buf1, g1, h1 * _SLAB_C, acc)

    acc = lax.fori_loop(0, ng * nh // 2, pair_step, jnp.zeros((_L,), jnp.float32))
    pltpu.make_async_copy(hs_src(0, 0), buf0, sem0).wait()
    accv[...] = acc
    pltpu.sync_copy(accv, out_hbm.at[wid])


def kernel(output, target):
    n, v = output.shape
    xt = output.T
    tgt = target.astype(jnp.int32)
    t2 = tgt.reshape(1, n)

    mesh = plsc.VectorSubcoreMesh(
        core_axis_name="c", subcore_axis_name="s", num_cores=_NC, num_subcores=_NS
    )
    sc_parts = pl.kernel(
        functools.partial(_sc_body, n, v),
        out_type=jax.ShapeDtypeStruct((_NW, _L), jnp.float32),
        mesh=mesh,
        scratch_types=[
            pltpu.VMEM((n,), jnp.int32),
            pltpu.VMEM((_SLAB_R, _SLAB_C), jnp.float32),
            pltpu.VMEM((_SLAB_R, _SLAB_C), jnp.float32),
            pltpu.VMEM((_L,), jnp.float32),
            pltpu.SemaphoreType.DMA,
            pltpu.SemaphoreType.DMA,
        ],
    )(xt, tgt)

    tc_out = pl.pallas_call(
        _tc_body,
        grid=(_VS // _VB,),
        in_specs=[
            pl.BlockSpec((1, n), lambda b: (0, 0)),
            pl.BlockSpec((_VB, n), lambda b: (b, 0)),
        ],
        out_specs=pl.BlockSpec(memory_space=pltpu.SMEM),
        out_shape=jax.ShapeDtypeStruct((1, 1), jnp.float32),
    )(t2, xt)

    return tc_out[0, 0] + jnp.sum(sc_parts)


# speedup vs baseline: 1.0669x; 1.0669x over previous
import functools
import math

import jax
import jax.numpy as jnp
from jax import lax
from jax.experimental import pallas as pl
from jax.experimental.pallas import tpu as pltpu
from jax.experimental.pallas import tpu_sc as plsc

_LS = 0.1
_V = 10000
_PAD = 0
_CONF = 1.0 - _LS
_SMOOTH = _LS / (_V - 2)
_C_ROW = (_V - 2) * _SMOOTH * math.log(_SMOOTH) + _CONF * math.log(_CONF)

_VB = 496
_VS = 7440
_NC, _NS, _L = 2, 16, 16
_NW = _NC * _NS


def _tc_body(t_ref, x_ref, o_ref):
    b = pl.program_id(0)
    x = x_ref[...]
    t = t_ref[...]
    colsum = jnp.sum(x, axis=0, keepdims=True)
    rows = jax.lax.broadcasted_iota(jnp.int32, x.shape, 0) + b * _VB
    tval = jnp.sum(jnp.where(rows == t, x, 0.0), axis=0, keepdims=True)
    contrib = -_SMOOTH * colsum - (_CONF - _SMOOTH) * tval
    head = jnp.where(b == 0, _C_ROW + _SMOOTH * x[0:1, :], 0.0)
    part = jnp.sum(jnp.where(t != _PAD, contrib + head, 0.0))

    @pl.when(b == 0)
    def _():
        o_ref[0, 0] = 0.0

    o_ref[0, 0] += part


_SLAB_R = 8
_SLAB_C = 4096


def _sc_body(n, v, xt_hbm, tgt_hbm, out_hbm, tvm, buf0, buf1, accv, sem0, sem1):
    rw = (v - _VS) // _NW
    ng = rw // _SLAB_R
    nh = n // _SLAB_C
    wid = lax.axis_index("s") * _NC + lax.axis_index("c")
    base = _VS + wid * rw
    pltpu.sync_copy(tgt_hbm, tvm)

    def hs_src(g, h):
        return xt_hbm.at[pl.ds(base + g * _SLAB_R, _SLAB_R),
                         pl.ds(h * _SLAB_C, _SLAB_C)]

    def compute(buf, g, co, acc):
        rbase = base + g * _SLAB_R

        def vec_step(k, kacc):
            sl = pl.ds(pl.multiple_of(k * _L, _L), _L)
            tk = tvm[pl.ds(pl.multiple_of(co + k * _L, _L), _L)]
            f0 = jnp.where(tk != _PAD, -_SMOOTH, 0.0)
            for rr in range(_SLAB_R):
                xv = buf[rr, sl]
                f = jnp.where(tk == rbase + rr, -_CONF, f0)
                kacc = kacc + xv * f
            return kacc

        return lax.fori_loop(0, _SLAB_C // _L, vec_step, acc)

    pltpu.async_copy(hs_src(0, 0), buf0, sem0)

    def pair_step(i, acc):
        g0 = (2 * i) // nh
        h0 = (2 * i) % nh
        g1 = (2 * i + 1) // nh
        h1 = (2 * i + 1) % nh
        g2 = jnp.minimum((2 * i + 2) // nh, ng - 1)
        h2 = (2 * i + 2) % nh
        pltpu.async_copy(hs_src(g1, h1), buf1, sem1)
        pltpu.make_async_copy(hs_src(g0, h0), buf0, sem0).wait()
        acc = compute(buf0, g0, h0 * _SLAB_C, acc)
        pltpu.async_copy(hs_src(g2, h2), buf0, sem0)
        pltpu.make_async_copy(hs_src(g1, h1), buf1, sem1).wait()
        return compute(buf1, g1, h1 * _SLAB_C, acc)

    acc = lax.fori_loop(0, ng * nh // 2, pair_step, jnp.zeros((_L,), jnp.float32))
    pltpu.make_async_copy(hs_src(0, 0), buf0, sem0).wait()
    accv[...] = acc
    pltpu.sync_copy(accv, out_hbm.at[wid])


def kernel(output, target):
    n, v = output.shape
    xt = output.T
    tgt = target.astype(jnp.int32)
    t2 = tgt.reshape(1, n)

    mesh = plsc.VectorSubcoreMesh(
        core_axis_name="c", subcore_axis_name="s", num_cores=_NC, num_subcores=_NS
    )
    sc_parts = pl.kernel(
        functools.partial(_sc_body, n, v),
        out_type=jax.ShapeDtypeStruct((_NW, _L), jnp.float32),
        mesh=mesh,
        scratch_types=[
            pltpu.VMEM((n,), jnp.int32),
            pltpu.VMEM((_SLAB_R, _SLAB_C), jnp.float32),
            pltpu.VMEM((_SLAB_R, _SLAB_C), jnp.float32),
            pltpu.VMEM((_L,), jnp.float32),
            pltpu.SemaphoreType.DMA,
            pltpu.SemaphoreType.DMA,
        ],
    )(xt, tgt)

    tc_out = pl.pallas_call(
        _tc_body,
        grid=(_VS // _VB,),
        in_specs=[
            pl.BlockSpec((1, n), lambda b: (0, 0)),
            pl.BlockSpec((_VB, n), lambda b: (b, 0)),
        ],
        out_specs=pl.BlockSpec(memory_space=pltpu.SMEM),
        out_shape=jax.ShapeDtypeStruct((1, 1), jnp.float32),
    )(t2, xt)

    return tc_out[0, 0] + jnp.sum(sc_parts)


# speedup vs baseline: 1.0725x; 1.0052x over previous
import functools
import math

import jax
import jax.numpy as jnp
from jax import lax
from jax.experimental import pallas as pl
from jax.experimental.pallas import tpu as pltpu
from jax.experimental.pallas import tpu_sc as plsc

_LS = 0.1
_V = 10000
_PAD = 0
_CONF = 1.0 - _LS
_SMOOTH = _LS / (_V - 2)
_C_ROW = (_V - 2) * _SMOOTH * math.log(_SMOOTH) + _CONF * math.log(_CONF)

_VB = 368
_VS = 8464
_NC, _NS, _L = 2, 16, 16
_NW = _NC * _NS


def _tc_body(t_ref, x_ref, o_ref):
    b = pl.program_id(0)
    x = x_ref[...]
    t = t_ref[...]
    colsum = jnp.sum(x, axis=0, keepdims=True)
    rows = jax.lax.broadcasted_iota(jnp.int32, x.shape, 0) + b * _VB
    tval = jnp.sum(jnp.where(rows == t, x, 0.0), axis=0, keepdims=True)
    contrib = -_SMOOTH * colsum - (_CONF - _SMOOTH) * tval
    head = jnp.where(b == 0, _C_ROW + _SMOOTH * x[0:1, :], 0.0)
    part = jnp.sum(jnp.where(t != _PAD, contrib + head, 0.0))

    @pl.when(b == 0)
    def _():
        o_ref[0, 0] = 0.0

    o_ref[0, 0] += part


_SLAB_R = 8
_SLAB_C = 4096


def _sc_body(n, v, xt_hbm, tgt_hbm, out_hbm, tvm, buf0, buf1, accv, sem0, sem1):
    rw = (v - _VS) // _NW
    ng = rw // _SLAB_R
    nh = n // _SLAB_C
    wid = lax.axis_index("s") * _NC + lax.axis_index("c")
    base = _VS + wid * rw
    pltpu.sync_copy(tgt_hbm, tvm)

    def hs_src(g, h):
        return xt_hbm.at[pl.ds(base + g * _SLAB_R, _SLAB_R),
                         pl.ds(h * _SLAB_C, _SLAB_C)]

    def compute(buf, g, co, acc):
        rbase = base + g * _SLAB_R

        def vec_step(k, kacc):
            sl = pl.ds(pl.multiple_of(k * _L, _L), _L)
            tk = tvm[pl.ds(pl.multiple_of(co + k * _L, _L), _L)]
            f0 = jnp.where(tk != _PAD, -_SMOOTH, 0.0)
            for rr in range(_SLAB_R):
                xv = buf[rr, sl]
                f = jnp.where(tk == rbase + rr, -_CONF, f0)
                kacc = kacc + xv * f
            return kacc

        return lax.fori_loop(0, _SLAB_C // _L, vec_step, acc)

    pltpu.async_copy(hs_src(0, 0), buf0, sem0)

    def pair_step(i, acc):
        g0 = (2 * i) // nh
        h0 = (2 * i) % nh
        g1 = (2 * i + 1) // nh
        h1 = (2 * i + 1) % nh
        g2 = jnp.minimum((2 * i + 2) // nh, ng - 1)
        h2 = (2 * i + 2) % nh
        pltpu.async_copy(hs_src(g1, h1), buf1, sem1)
        pltpu.make_async_copy(hs_src(g0, h0), buf0, sem0).wait()
        acc = compute(buf0, g0, h0 * _SLAB_C, acc)
        pltpu.async_copy(hs_src(g2, h2), buf0, sem0)
        pltpu.make_async_copy(hs_src(g1, h1), buf1, sem1).wait()
        return compute(buf1, g1, h1 * _SLAB_C, acc)

    acc = lax.fori_loop(0, ng * nh // 2, pair_step, jnp.zeros((_L,), jnp.float32))
    pltpu.make_async_copy(hs_src(0, 0), buf0, sem0).wait()
    accv[...] = acc
    pltpu.sync_copy(accv, out_hbm.at[wid])


def kernel(output, target):
    n, v = output.shape
    xt = output.T
    tgt = target.astype(jnp.int32)
    t2 = tgt.reshape(1, n)

    mesh = plsc.VectorSubcoreMesh(
        core_axis_name="c", subcore_axis_name="s", num_cores=_NC, num_subcores=_NS
    )
    sc_parts = pl.kernel(
        functools.partial(_sc_body, n, v),
        out_type=jax.ShapeDtypeStruct((_NW, _L), jnp.float32),
        mesh=mesh,
        scratch_types=[
            pltpu.VMEM((n,), jnp.int32),
            pltpu.VMEM((_SLAB_R, _SLAB_C), jnp.float32),
            pltpu.VMEM((_SLAB_R, _SLAB_C), jnp.float32),
            pltpu.VMEM((_L,), jnp.float32),
            pltpu.SemaphoreType.DMA,
            pltpu.SemaphoreType.DMA,
        ],
    )(xt, tgt)

    tc_out = pl.pallas_call(
        _tc_body,
        grid=(_VS // _VB,),
        in_specs=[
            pl.BlockSpec((1, n), lambda b: (0, 0)),
            pl.BlockSpec((_VB, n), lambda b: (b, 0)),
        ],
        out_specs=pl.BlockSpec(memory_space=pltpu.SMEM),
        out_shape=jax.ShapeDtypeStruct((1, 1), jnp.float32),
    )(t2, xt)

    return tc_out[0, 0] + jnp.sum(sc_parts)
